# baseline (speedup 1.0000x reference)
import functools

import jax
import jax.numpy as jnp
from jax import lax
from jax.experimental import pallas as pl
from jax.experimental.pallas import tpu as pltpu
from jax.experimental.pallas import tpu_sc as plsc

N_NODES = 10000
N_EDGES = 320000
IN_DIM = 128
HID_DIM = 256
OUT_DIM = 128

NPAD = 10240
ROWS_PER_TILE = NPAD // 16
CHUNK = 128
NCHUNK = 160
EDGES_PER_TILE = NCHUNK * CHUNK
EPAD = 16 * EDGES_PER_TILE
T_SPLIT = 80
GCH = 5
CCH = 64
CNCH = 320
CT = 160
BR = 1024

_sc_mesh = plsc.VectorSubcoreMesh(core_axis_name="c", subcore_axis_name="s")
_sc_params = pltpu.CompilerParams(needs_layout_passes=False)


@functools.partial(
    pl.kernel,
    out_type=(
        jax.ShapeDtypeStruct((NPAD, IN_DIM), jnp.float32),
        jax.ShapeDtypeStruct((NPAD,), jnp.float32),
    ),
    mesh=_sc_mesh,
    scratch_types=[
        pltpu.VMEM((GCH, CHUNK), jnp.int32),
        pltpu.VMEM((NCHUNK, CHUNK), jnp.int32),
        pltpu.VMEM((CHUNK, IN_DIM), jnp.float32),
        pltpu.VMEM((CHUNK, IN_DIM), jnp.float32),
        pltpu.VMEM((CHUNK, IN_DIM), jnp.float32),
        pltpu.VMEM((CHUNK,), jnp.float32),
        pltpu.VMEM((ROWS_PER_TILE,), jnp.float32),
        pltpu.VMEM_SHARED((NPAD,), jnp.float32),
        pltpu.SemaphoreType.DMA,
        pltpu.SemaphoreType.DMA,
        pltpu.SemaphoreType.DMA,
        pltpu.SemaphoreType.DMA,
        pltpu.SemaphoreType.DMA,
        pltpu.SemaphoreType.DMA,
        pltpu.SemaphoreType.DMA,
    ],
    compiler_params=_sc_params,
)
def _gather_deg_kernel(embed, ids3, dst3, ones, zvec, x_out, deg_out,
                       idx_all, didx_all, rows0, rows1, rows2, ones_v, zvec_v,
                       deg_sh, g0, g1, g2, s0, s1, s2, dsem):
    c = lax.axis_index("c")
    s = lax.axis_index("s")
    rows = [rows0, rows1, rows2]
    gsem = [g0, g1, g2]
    ssem = [s0, s1, s2]

    @pl.when(c == 1)
    def _():
        pltpu.sync_copy(zvec, zvec_v)
        pltpu.sync_copy(zvec_v, deg_sh.at[pl.ds(s * ROWS_PER_TILE, ROWS_PER_TILE)])

    plsc.subcore_barrier()

    @pl.when(c == 0)
    def _():
        pltpu.sync_copy(ids3.at[s], idx_all)
        for j in range(2):
            pltpu.async_copy(embed.at[idx_all.at[j]], rows[j], gsem[j])
        for j in range(GCH):
            b = j % 3
            pltpu.make_async_copy(embed.at[pl.ds(0, CHUNK)], rows[b], gsem[b]).wait()
            pltpu.async_copy(
                rows[b], x_out.at[pl.ds(s * ROWS_PER_TILE + j * CHUNK, CHUNK)],
                ssem[b])
            if j + 2 < GCH:
                nb = (j + 2) % 3
                if j >= 1:
                    pltpu.make_async_copy(
                        embed.at[pl.ds(0, CHUNK)], rows[nb], ssem[nb]).wait()
                pltpu.async_copy(embed.at[idx_all.at[j + 2]], rows[nb], gsem[nb])
        for j in range(3, 6):
            b = j % 3
            pltpu.make_async_copy(embed.at[pl.ds(0, CHUNK)], rows[b], ssem[b]).wait()

    @pl.when(c == 1)
    def _():
        pltpu.sync_copy(ones, ones_v)
        pltpu.sync_copy(dst3.at[s], didx_all)

        @pl.loop(0, NCHUNK)
        def _(j):
            pltpu.async_copy(ones_v, deg_sh.at[didx_all.at[j]], dsem, add=True)

        pltpu.make_async_copy(dst3.at[s], didx_all, dsem).wait()

    plsc.subcore_barrier()

    @pl.when(c == 1)
    def _():
        off = s * ROWS_PER_TILE
        pltpu.sync_copy(deg_sh.at[pl.ds(off, ROWS_PER_TILE)],
                        deg_out.at[pl.ds(off, ROWS_PER_TILE)])


@functools.partial(
    pl.kernel,
    out_type=(
        jax.ShapeDtypeStruct((NPAD, 128), jnp.float32),
        jax.ShapeDtypeStruct((NPAD, 128), jnp.float32),
        jax.ShapeDtypeStruct((NPAD,), jnp.float32),
        jax.ShapeDtypeStruct((NPAD,), jnp.float32),
    ),
    mesh=_sc_mesh,
    scratch_types=[
        pltpu.VMEM((8, CCH), jnp.int32),
        pltpu.VMEM((8, CCH), jnp.int32),
        pltpu.VMEM((CCH, 128), jnp.float32),
        pltpu.VMEM((CCH, 128), jnp.float32),
        pltpu.VMEM((CCH, 128), jnp.float32),
        pltpu.VMEM((CCH, 128), jnp.float32),
        pltpu.VMEM((CCH,), jnp.float32),
        pltpu.VMEM((CCH,), jnp.float32),
        pltpu.VMEM((ROWS_PER_TILE,), jnp.float32),
        pltpu.VMEM_SHARED((NPAD, 128), jnp.float32),
        pltpu.VMEM_SHARED((NPAD,), jnp.float32),
        [pltpu.SemaphoreType.DMA] * 8,
        [pltpu.SemaphoreType.DMA] * 4,
        [pltpu.SemaphoreType.DMA] * 4,
        [pltpu.SemaphoreType.DMA] * 2,
        [pltpu.SemaphoreType.DMA] * 2,
    ],
    compiler_params=_sc_params,
)
def _agg_kernel(y0, y1, src3, dst3, dinv, zrows, zvec,
                agg0, agg1, t0, t1,
                sidx, didx, rows0, rows1, rows2, rows3,
                dvals0, dvals1, zvec_v,
                agg_sh, t_sh, isem, gsem, ssem, dgsem, tsem):
    c = lax.axis_index("c")
    s = lax.axis_index("s")
    off = s * ROWS_PER_TILE
    rows = [rows0, rows1, rows2, rows3]
    dvals = [dvals0, dvals1]

    pltpu.sync_copy(zrows, rows0)
    for k in range(ROWS_PER_TILE // CCH):
        pltpu.sync_copy(rows0, agg_sh.at[pl.ds(off + k * CCH, CCH)])
    pltpu.sync_copy(zvec, zvec_v)
    pltpu.sync_copy(zvec_v, t_sh.at[pl.ds(off, ROWS_PER_TILE)])

    plsc.subcore_barrier()

    def fire_idx(g, b8):
        pltpu.async_copy(src3.at[s, g], sidx.at[b8], isem[b8])
        pltpu.async_copy(dst3.at[s, g], didx.at[b8], isem[b8])

    def wait_idx(b8):
        pltpu.make_async_copy(src3.at[s, 0], sidx.at[b8], isem[b8]).wait()
        pltpu.make_async_copy(src3.at[s, 0], didx.at[b8], isem[b8]).wait()

    def fire_gather(b8, b4):
        @pl.when(c == 0)
        def _():
            pltpu.async_copy(y0.at[sidx.at[b8]], rows[b4], gsem[b4])

        @pl.when(c == 1)
        def _():
            pltpu.async_copy(y1.at[sidx.at[b8]], rows[b4], gsem[b4])

    def wait_rows(sem):
        pltpu.make_async_copy(y0.at[pl.ds(0, CCH)], rows0, sem).wait()

    def wait_dvals(sem, p):
        pltpu.make_async_copy(zvec.at[pl.ds(0, CCH)], dvals[p], sem).wait()

    def do_t(g):
        return jnp.where(c == 0, g < CT, g >= CT)

    for b in range(4):
        fire_idx(b, b)
    for b in range(3):
        wait_idx(b)
        fire_gather(b, b)

    @pl.when(c == 0)
    def _():
        pltpu.async_copy(dinv.at[didx.at[0]], dvals[0], dgsem[0])

    @pl.loop(0, CNCH // 8)
    def _(G):
        for u in range(8):
            g = G * 8 + u
            b4 = u % 4
            p = u % 2
            pn = (u + 1) % 2

            wait_rows(gsem[b4])
            pltpu.async_copy(rows[b4], agg_sh.at[didx.at[u]], ssem[b4],
                             add=True)

            @pl.when(do_t(g))
            def _():
                wait_dvals(dgsem[p], p)
                pltpu.async_copy(dvals[p], t_sh.at[sidx.at[u]], tsem[p],
                                 add=True)

            @pl.when(g + 1 < CNCH)
            def _():
                @pl.when(do_t(g + 1))
                def _():
                    @pl.when(jnp.where(c == 0, g >= 1, g >= CT + 1))
                    def _():
                        wait_dvals(tsem[pn], pn)

                    pltpu.async_copy(dinv.at[didx.at[(u + 1) % 8]], dvals[pn],
                                     dgsem[pn])

            @pl.when(g + 3 < CNCH)
            def _():
                wait_idx((u + 3) % 8)

                @pl.when(g >= 1)
                def _():
                    wait_rows(ssem[(u + 3) % 4])

                fire_gather((u + 3) % 8, (u + 3) % 4)

            @pl.when(g + 4 < CNCH)
            def _():
                fire_idx(g + 4, (u + 4) % 8)

    for b in range(4):
        wait_rows(ssem[b])
    wait_dvals(tsem[0], 0)
    wait_dvals(tsem[1], 1)

    plsc.subcore_barrier()

    @pl.when(c == 0)
    def _():
        for k in range(ROWS_PER_TILE // CHUNK):
            o = off + k * CHUNK
            pltpu.sync_copy(agg_sh.at[pl.ds(o, CHUNK)], agg0.at[pl.ds(o, CHUNK)])
        pltpu.sync_copy(t_sh.at[pl.ds(off, ROWS_PER_TILE)],
                        t0.at[pl.ds(off, ROWS_PER_TILE)])

    @pl.when(c == 1)
    def _():
        for k in range(ROWS_PER_TILE // CHUNK):
            o = off + k * CHUNK
            pltpu.sync_copy(agg_sh.at[pl.ds(o, CHUNK)], agg1.at[pl.ds(o, CHUNK)])
        pltpu.sync_copy(t_sh.at[pl.ds(off, ROWS_PER_TILE)],
                        t1.at[pl.ds(off, ROWS_PER_TILE)])


def _layer1_body(x_ref, w_ref, deg_ref, y0_ref, y1_ref, dinv_ref):
    i = pl.program_id(0)
    rows = lax.broadcasted_iota(jnp.int32, (BR, 1), 0) + i * BR
    dinv = jnp.where(rows < N_NODES, lax.rsqrt(deg_ref[...] + 1.0), 0.0)
    h = jnp.dot(x_ref[...], w_ref[...], preferred_element_type=jnp.float32,
                precision=lax.Precision.HIGHEST)
    y = h * dinv
    y0_ref[...] = y[:, :128]
    y1_ref[...] = y[:, 128:]
    dinv_ref[...] = dinv


def _layer1(x, w_in, deg_col):
    return pl.pallas_call(
        _layer1_body,
        grid=(NPAD // BR,),
        in_specs=[
            pl.BlockSpec((BR, IN_DIM), lambda i: (i, 0)),
            pl.BlockSpec((IN_DIM, HID_DIM), lambda i: (0, 0)),
            pl.BlockSpec((BR, 1), lambda i: (i, 0)),
        ],
        out_specs=[
            pl.BlockSpec((BR, 128), lambda i: (i, 0)),
            pl.BlockSpec((BR, 128), lambda i: (i, 0)),
            pl.BlockSpec((BR, 1), lambda i: (i, 0)),
        ],
        out_shape=[
            jax.ShapeDtypeStruct((NPAD, 128), jnp.float32),
            jax.ShapeDtypeStruct((NPAD, 128), jnp.float32),
            jax.ShapeDtypeStruct((NPAD, 1), jnp.float32),
        ],
    )(x, w_in, deg_col)


def _final_body(agg0_ref, agg1_ref, y0_ref, y1_ref, dinv_ref, t0_ref, t1_ref,
                b_in_ref, w_out_ref, b_out_ref, out_ref, acc):
    i = pl.program_id(0)
    dinv = dinv_ref[...]
    cvec = dinv * (dinv + t0_ref[...] + t1_ref[...])
    pre = jnp.concatenate(
        [agg0_ref[...].astype(jnp.float32) + y0_ref[...].astype(jnp.float32),
         agg1_ref[...].astype(jnp.float32) + y1_ref[...].astype(jnp.float32)],
        axis=1)
    x1 = jnp.maximum(pre * dinv + b_in_ref[...], 0.0)
    v = jnp.sum(x1 * cvec, axis=0, keepdims=True)

    @pl.when(i == 0)
    def _():
        acc[...] = v

    @pl.when(i > 0)
    def _():
        acc[...] += v

    @pl.when(i == NPAD // BR - 1)
    def _():
        out_ref[...] = (
            jnp.dot(acc[...], w_out_ref[...], preferred_element_type=jnp.float32,
                    precision=lax.Precision.HIGHEST) * (1.0 / N_NODES)
            + b_out_ref[...])


def _final(agg0, agg1, y0, y1, dinv_col, t0_col, t1_col, b_in, w_out, b_out):
    return pl.pallas_call(
        _final_body,
        grid=(NPAD // BR,),
        in_specs=[
            pl.BlockSpec((BR, 128), lambda i: (i, 0)),
            pl.BlockSpec((BR, 128), lambda i: (i, 0)),
            pl.BlockSpec((BR, 128), lambda i: (i, 0)),
            pl.BlockSpec((BR, 128), lambda i: (i, 0)),
            pl.BlockSpec((BR, 1), lambda i: (i, 0)),
            pl.BlockSpec((BR, 1), lambda i: (i, 0)),
            pl.BlockSpec((BR, 1), lambda i: (i, 0)),
            pl.BlockSpec((1, HID_DIM), lambda i: (0, 0)),
            pl.BlockSpec((HID_DIM, OUT_DIM), lambda i: (0, 0)),
            pl.BlockSpec((1, OUT_DIM), lambda i: (0, 0)),
        ],
        out_specs=pl.BlockSpec((1, OUT_DIM), lambda i: (0, 0)),
        out_shape=jax.ShapeDtypeStruct((1, OUT_DIM), jnp.float32),
        scratch_shapes=[pltpu.VMEM((1, HID_DIM), jnp.float32)],
    )(agg0, agg1, y0, y1, dinv_col, t0_col, t1_col, b_in, w_out, b_out)


def kernel(node_ids, edge_index, embed, W_in, b_in, W_out, b_out):
    src = edge_index[0].astype(jnp.int32)
    dst = edge_index[1].astype(jnp.int32)
    ids3 = jnp.concatenate(
        [node_ids.astype(jnp.int32), jnp.zeros((NPAD - N_NODES,), jnp.int32)]
    ).reshape(16, GCH, CHUNK)
    src_pad = jnp.concatenate([src, jnp.zeros((EPAD - N_EDGES,), jnp.int32)])
    dst_pad = jnp.concatenate(
        [dst, jnp.full((EPAD - N_EDGES,), NPAD - 1, jnp.int32)])
    ones = jnp.ones((CHUNK,), jnp.float32)
    zrows = jnp.zeros((CCH, 128), jnp.float32)
    zvec = jnp.zeros((ROWS_PER_TILE,), jnp.float32)

    x, deg = _gather_deg_kernel(
        embed, ids3, dst_pad.reshape(16, NCHUNK, CHUNK), ones, zvec)
    y0, y1, dinv_col = _layer1(x, W_in, deg.reshape(NPAD, 1))
    agg0, agg1, t0, t1 = _agg_kernel(
        y0, y1, src_pad.reshape(16, CNCH, CCH), dst_pad.reshape(16, CNCH, CCH),
        dinv_col.reshape(NPAD), zrows, zvec)
    out = _final(agg0, agg1, y0, y1, dinv_col,
                 t0.reshape(NPAD, 1), t1.reshape(NPAD, 1),
                 b_in.reshape(1, HID_DIM), W_out, b_out.reshape(1, OUT_DIM))
    return out.reshape(OUT_DIM)


# speedup vs baseline: 1.0689x; 1.0689x over previous
import functools

import jax
import jax.numpy as jnp
from jax import lax
from jax.experimental import pallas as pl
from jax.experimental.pallas import tpu as pltpu
from jax.experimental.pallas import tpu_sc as plsc

N_NODES = 10000
N_EDGES = 320000
IN_DIM = 128
HID_DIM = 256
OUT_DIM = 128

NPAD = 10240
ROWS_PER_TILE = NPAD // 16
CHUNK = 128
NCHUNK = 160
EDGES_PER_TILE = NCHUNK * CHUNK
EPAD = 16 * EDGES_PER_TILE
T_SPLIT = 80
GCH = 5
BR = 1024

_sc_mesh = plsc.VectorSubcoreMesh(core_axis_name="c", subcore_axis_name="s")
_sc_params = pltpu.CompilerParams(needs_layout_passes=False)


@functools.partial(
    pl.kernel,
    out_type=(
        jax.ShapeDtypeStruct((NPAD, IN_DIM), jnp.float32),
        jax.ShapeDtypeStruct((NPAD,), jnp.float32),
    ),
    mesh=_sc_mesh,
    scratch_types=[
        pltpu.VMEM((GCH, CHUNK), jnp.int32),
        pltpu.VMEM((NCHUNK, CHUNK), jnp.int32),
        pltpu.VMEM((CHUNK, IN_DIM), jnp.float32),
        pltpu.VMEM((CHUNK, IN_DIM), jnp.float32),
        pltpu.VMEM((CHUNK, IN_DIM), jnp.float32),
        pltpu.VMEM((CHUNK,), jnp.float32),
        pltpu.VMEM((ROWS_PER_TILE,), jnp.float32),
        pltpu.VMEM_SHARED((NPAD,), jnp.float32),
        pltpu.SemaphoreType.DMA,
        pltpu.SemaphoreType.DMA,
        pltpu.SemaphoreType.DMA,
        pltpu.SemaphoreType.DMA,
        pltpu.SemaphoreType.DMA,
        pltpu.SemaphoreType.DMA,
        pltpu.SemaphoreType.DMA,
    ],
    compiler_params=_sc_params,
)
def _gather_deg_kernel(embed, ids3, dst3, ones, zvec, x_out, deg_out,
                       idx_all, didx_all, rows0, rows1, rows2, ones_v, zvec_v,
                       deg_sh, g0, g1, g2, s0, s1, s2, dsem):
    c = lax.axis_index("c")
    s = lax.axis_index("s")
    rows = [rows0, rows1, rows2]
    gsem = [g0, g1, g2]
    ssem = [s0, s1, s2]

    @pl.when(c == 1)
    def _():
        pltpu.sync_copy(zvec, zvec_v)
        pltpu.sync_copy(zvec_v, deg_sh.at[pl.ds(s * ROWS_PER_TILE, ROWS_PER_TILE)])

    plsc.subcore_barrier()

    @pl.when(c == 0)
    def _():
        pltpu.sync_copy(ids3.at[s], idx_all)
        for j in range(2):
            pltpu.async_copy(embed.at[idx_all.at[j]], rows[j], gsem[j])
        for j in range(GCH):
            b = j % 3
            pltpu.make_async_copy(embed.at[pl.ds(0, CHUNK)], rows[b], gsem[b]).wait()
            pltpu.async_copy(
                rows[b], x_out.at[pl.ds(s * ROWS_PER_TILE + j * CHUNK, CHUNK)],
                ssem[b])
            if j + 2 < GCH:
                nb = (j + 2) % 3
                if j >= 1:
                    pltpu.make_async_copy(
                        embed.at[pl.ds(0, CHUNK)], rows[nb], ssem[nb]).wait()
                pltpu.async_copy(embed.at[idx_all.at[j + 2]], rows[nb], gsem[nb])
        for j in range(3, 6):
            b = j % 3
            pltpu.make_async_copy(embed.at[pl.ds(0, CHUNK)], rows[b], ssem[b]).wait()

    @pl.when(c == 1)
    def _():
        pltpu.sync_copy(ones, ones_v)
        pltpu.sync_copy(dst3.at[s], didx_all)

        @pl.loop(0, NCHUNK)
        def _(j):
            pltpu.async_copy(ones_v, deg_sh.at[didx_all.at[j]], dsem, add=True)

        pltpu.make_async_copy(dst3.at[s], didx_all, dsem).wait()

    plsc.subcore_barrier()

    @pl.when(c == 1)
    def _():
        off = s * ROWS_PER_TILE
        pltpu.sync_copy(deg_sh.at[pl.ds(off, ROWS_PER_TILE)],
                        deg_out.at[pl.ds(off, ROWS_PER_TILE)])


@functools.partial(
    pl.kernel,
    out_type=(
        jax.ShapeDtypeStruct((NPAD, 128), jnp.float32),
        jax.ShapeDtypeStruct((NPAD, 128), jnp.float32),
        jax.ShapeDtypeStruct((NPAD,), jnp.float32),
        jax.ShapeDtypeStruct((NPAD,), jnp.float32),
    ),
    mesh=_sc_mesh,
    scratch_types=[
        pltpu.VMEM((8, CHUNK), jnp.int32),
        pltpu.VMEM((8, CHUNK), jnp.int32),
        pltpu.VMEM((CHUNK, 128), jnp.float32),
        pltpu.VMEM((CHUNK, 128), jnp.float32),
        pltpu.VMEM((CHUNK,), jnp.float32),
        pltpu.VMEM((CHUNK,), jnp.float32),
        pltpu.VMEM((ROWS_PER_TILE,), jnp.float32),
        pltpu.VMEM_SHARED((NPAD, 128), jnp.float32),
        pltpu.VMEM_SHARED((NPAD,), jnp.float32),
        [pltpu.SemaphoreType.DMA] * 8,
        [pltpu.SemaphoreType.DMA] * 2,
        [pltpu.SemaphoreType.DMA] * 2,
        [pltpu.SemaphoreType.DMA] * 2,
        [pltpu.SemaphoreType.DMA] * 2,
    ],
    compiler_params=_sc_params,
)
def _agg_kernel(y0, y1, src3, dst3, dinv, zvec,
                agg0, agg1, t0, t1,
                sidx, didx, rows0, rows1, dvals0, dvals1, zvec_v,
                agg_sh, t_sh, isem, gsem, ssem, dgsem, tsem):
    c = lax.axis_index("c")
    s = lax.axis_index("s")
    off = s * ROWS_PER_TILE
    rows = [rows0, rows1]
    dvals = [dvals0, dvals1]

    for k in range(ROWS_PER_TILE // CHUNK):
        sl = pl.ds(off + k * CHUNK, CHUNK)

        @pl.when(c == 0)
        def _():
            pltpu.sync_copy(y0.at[sl], rows0)
            pltpu.sync_copy(rows0, agg_sh.at[sl])

        @pl.when(c == 1)
        def _():
            pltpu.sync_copy(y1.at[sl], rows0)
            pltpu.sync_copy(rows0, agg_sh.at[sl])

    pltpu.sync_copy(zvec, zvec_v)
    pltpu.sync_copy(zvec_v, t_sh.at[pl.ds(off, ROWS_PER_TILE)])

    plsc.subcore_barrier()

    def fire_idx(g, b8):
        pltpu.async_copy(src3.at[s, g], sidx.at[b8], isem[b8])
        pltpu.async_copy(dst3.at[s, g], didx.at[b8], isem[b8])

    def wait_idx(b8):
        pltpu.make_async_copy(src3.at[s, 0], sidx.at[b8], isem[b8]).wait()
        pltpu.make_async_copy(src3.at[s, 0], didx.at[b8], isem[b8]).wait()

    def fire_gather(b8, b2):
        @pl.when(c == 0)
        def _():
            pltpu.async_copy(y0.at[sidx.at[b8]], rows[b2], gsem[b2])

        @pl.when(c == 1)
        def _():
            pltpu.async_copy(y1.at[sidx.at[b8]], rows[b2], gsem[b2])

    def wait_rows(sem):
        pltpu.make_async_copy(y0.at[pl.ds(0, CHUNK)], rows0, sem).wait()

    def wait_dvals(sem, p):
        pltpu.make_async_copy(zvec.at[pl.ds(0, CHUNK)], dvals[p], sem).wait()

    def do_t(g):
        return jnp.where(c == 0, g < T_SPLIT, g >= T_SPLIT)

    fire_idx(0, 0)
    fire_idx(1, 1)
    wait_idx(0)
    fire_gather(0, 0)

    @pl.when(c == 0)
    def _():
        pltpu.async_copy(dinv.at[didx.at[0]], dvals[0], dgsem[0])

    @pl.loop(0, NCHUNK // 8)
    def _(G):
        for u in range(8):
            g = G * 8 + u
            b2 = u % 2
            p = u % 2
            pn = (u + 1) % 2
            nb8 = (u + 1) % 8

            wait_rows(gsem[b2])
            pltpu.async_copy(rows[b2], agg_sh.at[didx.at[u]], ssem[b2],
                             add=True)

            @pl.when(do_t(g))
            def _():
                wait_dvals(dgsem[p], p)
                pltpu.async_copy(dvals[p], t_sh.at[sidx.at[u]], tsem[p],
                                 add=True)

            @pl.when(g + 1 < NCHUNK)
            def _():
                wait_idx(nb8)

                @pl.when(do_t(g + 1))
                def _():
                    @pl.when(jnp.where(c == 0, g >= 1, g >= T_SPLIT + 1))
                    def _():
                        wait_dvals(tsem[pn], pn)

                    pltpu.async_copy(dinv.at[didx.at[nb8]], dvals[pn],
                                     dgsem[pn])

                @pl.when(g >= 1)
                def _():
                    wait_rows(ssem[pn])

                fire_gather(nb8, pn)

            @pl.when(g + 2 < NCHUNK)
            def _():
                fire_idx(g + 2, (u + 2) % 8)

    wait_rows(ssem[0])
    wait_rows(ssem[1])
    wait_dvals(tsem[0], 0)
    wait_dvals(tsem[1], 1)

    plsc.subcore_barrier()

    @pl.when(c == 0)
    def _():
        for k in range(ROWS_PER_TILE // CHUNK):
            o = off + k * CHUNK
            pltpu.sync_copy(agg_sh.at[pl.ds(o, CHUNK)], agg0.at[pl.ds(o, CHUNK)])
        pltpu.sync_copy(t_sh.at[pl.ds(off, ROWS_PER_TILE)],
                        t0.at[pl.ds(off, ROWS_PER_TILE)])

    @pl.when(c == 1)
    def _():
        for k in range(ROWS_PER_TILE // CHUNK):
            o = off + k * CHUNK
            pltpu.sync_copy(agg_sh.at[pl.ds(o, CHUNK)], agg1.at[pl.ds(o, CHUNK)])
        pltpu.sync_copy(t_sh.at[pl.ds(off, ROWS_PER_TILE)],
                        t1.at[pl.ds(off, ROWS_PER_TILE)])


def _layer1_body(x_ref, w_ref, deg_ref, y0_ref, y1_ref, dinv_ref):
    i = pl.program_id(0)
    rows = lax.broadcasted_iota(jnp.int32, (BR, 1), 0) + i * BR
    dinv = jnp.where(rows < N_NODES, lax.rsqrt(deg_ref[...] + 1.0), 0.0)
    h = jnp.dot(x_ref[...], w_ref[...], preferred_element_type=jnp.float32,
                precision=lax.Precision.HIGHEST)
    y = h * dinv
    y0_ref[...] = y[:, :128]
    y1_ref[...] = y[:, 128:]
    dinv_ref[...] = dinv


def _layer1(x, w_in, deg_col):
    return pl.pallas_call(
        _layer1_body,
        grid=(NPAD // BR,),
        in_specs=[
            pl.BlockSpec((BR, IN_DIM), lambda i: (i, 0)),
            pl.BlockSpec((IN_DIM, HID_DIM), lambda i: (0, 0)),
            pl.BlockSpec((BR, 1), lambda i: (i, 0)),
        ],
        out_specs=[
            pl.BlockSpec((BR, 128), lambda i: (i, 0)),
            pl.BlockSpec((BR, 128), lambda i: (i, 0)),
            pl.BlockSpec((BR, 1), lambda i: (i, 0)),
        ],
        out_shape=[
            jax.ShapeDtypeStruct((NPAD, 128), jnp.float32),
            jax.ShapeDtypeStruct((NPAD, 128), jnp.float32),
            jax.ShapeDtypeStruct((NPAD, 1), jnp.float32),
        ],
    )(x, w_in, deg_col)


def _final_body(agg0_ref, agg1_ref, dinv_ref, t0_ref, t1_ref,
                b_in_ref, w_out_ref, b_out_ref, out_ref, acc):
    i = pl.program_id(0)
    dinv = dinv_ref[...]
    cvec = dinv * (dinv + t0_ref[...] + t1_ref[...])
    pre = jnp.concatenate([agg0_ref[...], agg1_ref[...]], axis=1)
    x1 = jnp.maximum(pre * dinv + b_in_ref[...], 0.0)
    v = jnp.sum(x1 * cvec, axis=0, keepdims=True)

    @pl.when(i == 0)
    def _():
        acc[...] = v

    @pl.when(i > 0)
    def _():
        acc[...] += v

    @pl.when(i == NPAD // BR - 1)
    def _():
        out_ref[...] = (
            jnp.dot(acc[...], w_out_ref[...], preferred_element_type=jnp.float32,
                    precision=lax.Precision.HIGHEST) * (1.0 / N_NODES)
            + b_out_ref[...])


def _final(agg0, agg1, dinv_col, t0_col, t1_col, b_in, w_out, b_out):
    return pl.pallas_call(
        _final_body,
        grid=(NPAD // BR,),
        in_specs=[
            pl.BlockSpec((BR, 128), lambda i: (i, 0)),
            pl.BlockSpec((BR, 128), lambda i: (i, 0)),
            pl.BlockSpec((BR, 1), lambda i: (i, 0)),
            pl.BlockSpec((BR, 1), lambda i: (i, 0)),
            pl.BlockSpec((BR, 1), lambda i: (i, 0)),
            pl.BlockSpec((1, HID_DIM), lambda i: (0, 0)),
            pl.BlockSpec((HID_DIM, OUT_DIM), lambda i: (0, 0)),
            pl.BlockSpec((1, OUT_DIM), lambda i: (0, 0)),
        ],
        out_specs=pl.BlockSpec((1, OUT_DIM), lambda i: (0, 0)),
        out_shape=jax.ShapeDtypeStruct((1, OUT_DIM), jnp.float32),
        scratch_shapes=[pltpu.VMEM((1, HID_DIM), jnp.float32)],
    )(agg0, agg1, dinv_col, t0_col, t1_col, b_in, w_out, b_out)


def kernel(node_ids, edge_index, embed, W_in, b_in, W_out, b_out):
    src = edge_index[0].astype(jnp.int32)
    dst = edge_index[1].astype(jnp.int32)
    ids3 = jnp.concatenate(
        [node_ids.astype(jnp.int32), jnp.zeros((NPAD - N_NODES,), jnp.int32)]
    ).reshape(16, GCH, CHUNK)
    src3 = jnp.concatenate(
        [src, jnp.zeros((EPAD - N_EDGES,), jnp.int32)]).reshape(16, NCHUNK, CHUNK)
    dst3 = jnp.concatenate(
        [dst, jnp.full((EPAD - N_EDGES,), NPAD - 1, jnp.int32)]
    ).reshape(16, NCHUNK, CHUNK)
    ones = jnp.ones((CHUNK,), jnp.float32)
    zvec = jnp.zeros((ROWS_PER_TILE,), jnp.float32)

    x, deg = _gather_deg_kernel(embed, ids3, dst3, ones, zvec)
    y0, y1, dinv_col = _layer1(x, W_in, deg.reshape(NPAD, 1))
    agg0, agg1, t0, t1 = _agg_kernel(
        y0, y1, src3, dst3, dinv_col.reshape(NPAD), zvec)
    out = _final(agg0, agg1, dinv_col,
                 t0.reshape(NPAD, 1), t1.reshape(NPAD, 1),
                 b_in.reshape(1, HID_DIM), W_out, b_out.reshape(1, OUT_DIM))
    return out.reshape(OUT_DIM)


# speedup vs baseline: 1.0753x; 1.0059x over previous
import functools

import jax
import jax.numpy as jnp
from jax import lax
from jax.experimental import pallas as pl
from jax.experimental.pallas import tpu as pltpu
from jax.experimental.pallas import tpu_sc as plsc

N_NODES = 10000
N_EDGES = 320000
IN_DIM = 128
HID_DIM = 256
OUT_DIM = 128

NPAD = 10240
ROWS_PER_TILE = NPAD // 16
CHUNK = 128
NCHUNK = 160
EDGES_PER_TILE = NCHUNK * CHUNK
EPAD = 16 * EDGES_PER_TILE
T_SPLIT = 80
GCH = 5
BR = 1024

_sc_mesh = plsc.VectorSubcoreMesh(core_axis_name="c", subcore_axis_name="s")
_sc_params = pltpu.CompilerParams(needs_layout_passes=False)


@functools.partial(
    pl.kernel,
    out_type=(
        jax.ShapeDtypeStruct((NPAD, IN_DIM), jnp.float32),
        jax.ShapeDtypeStruct((NPAD,), jnp.float32),
    ),
    mesh=_sc_mesh,
    scratch_types=[
        pltpu.VMEM((GCH, CHUNK), jnp.int32),
        pltpu.VMEM((NCHUNK, CHUNK), jnp.int32),
        pltpu.VMEM((CHUNK, IN_DIM), jnp.float32),
        pltpu.VMEM((CHUNK, IN_DIM), jnp.float32),
        pltpu.VMEM((CHUNK, IN_DIM), jnp.float32),
        pltpu.VMEM((CHUNK,), jnp.float32),
        pltpu.VMEM((ROWS_PER_TILE,), jnp.float32),
        pltpu.VMEM_SHARED((NPAD,), jnp.float32),
        pltpu.SemaphoreType.DMA,
        pltpu.SemaphoreType.DMA,
        pltpu.SemaphoreType.DMA,
        pltpu.SemaphoreType.DMA,
        pltpu.SemaphoreType.DMA,
        pltpu.SemaphoreType.DMA,
        pltpu.SemaphoreType.DMA,
    ],
    compiler_params=_sc_params,
)
def _gather_deg_kernel(embed, ids3, dst3, ones, zvec, x_out, deg_out,
                       idx_all, didx_all, rows0, rows1, rows2, ones_v, zvec_v,
                       deg_sh, g0, g1, g2, s0, s1, s2, dsem):
    c = lax.axis_index("c")
    s = lax.axis_index("s")
    rows = [rows0, rows1, rows2]
    gsem = [g0, g1, g2]
    ssem = [s0, s1, s2]

    @pl.when(c == 1)
    def _():
        pltpu.sync_copy(zvec, zvec_v)
        pltpu.sync_copy(zvec_v, deg_sh.at[pl.ds(s * ROWS_PER_TILE, ROWS_PER_TILE)])

    plsc.subcore_barrier()

    @pl.when(c == 0)
    def _():
        pltpu.sync_copy(ids3.at[s], idx_all)
        for j in range(2):
            pltpu.async_copy(embed.at[idx_all.at[j]], rows[j], gsem[j])
        for j in range(GCH):
            b = j % 3
            pltpu.make_async_copy(embed.at[pl.ds(0, CHUNK)], rows[b], gsem[b]).wait()
            pltpu.async_copy(
                rows[b], x_out.at[pl.ds(s * ROWS_PER_TILE + j * CHUNK, CHUNK)],
                ssem[b])
            if j + 2 < GCH:
                nb = (j + 2) % 3
                if j >= 1:
                    pltpu.make_async_copy(
                        embed.at[pl.ds(0, CHUNK)], rows[nb], ssem[nb]).wait()
                pltpu.async_copy(embed.at[idx_all.at[j + 2]], rows[nb], gsem[nb])
        for j in range(3, 6):
            b = j % 3
            pltpu.make_async_copy(embed.at[pl.ds(0, CHUNK)], rows[b], ssem[b]).wait()

    @pl.when(c == 1)
    def _():
        pltpu.sync_copy(ones, ones_v)
        pltpu.sync_copy(dst3.at[s], didx_all)

        @pl.loop(0, NCHUNK)
        def _(j):
            pltpu.async_copy(ones_v, deg_sh.at[didx_all.at[j]], dsem, add=True)

        pltpu.make_async_copy(dst3.at[s], didx_all, dsem).wait()

    plsc.subcore_barrier()

    @pl.when(c == 1)
    def _():
        off = s * ROWS_PER_TILE
        pltpu.sync_copy(deg_sh.at[pl.ds(off, ROWS_PER_TILE)],
                        deg_out.at[pl.ds(off, ROWS_PER_TILE)])


@functools.partial(
    pl.kernel,
    out_type=(
        jax.ShapeDtypeStruct((NPAD, 128), jnp.float32),
        jax.ShapeDtypeStruct((NPAD, 128), jnp.float32),
        jax.ShapeDtypeStruct((NPAD,), jnp.float32),
        jax.ShapeDtypeStruct((NPAD,), jnp.float32),
    ),
    mesh=_sc_mesh,
    scratch_types=[
        pltpu.VMEM((8, CHUNK), jnp.int32),
        pltpu.VMEM((8, CHUNK), jnp.int32),
        pltpu.VMEM((CHUNK, 128), jnp.float32),
        pltpu.VMEM((CHUNK, 128), jnp.float32),
        pltpu.VMEM((CHUNK,), jnp.float32),
        pltpu.VMEM((CHUNK,), jnp.float32),
        pltpu.VMEM((ROWS_PER_TILE,), jnp.float32),
        pltpu.VMEM_SHARED((NPAD, 128), jnp.float32),
        pltpu.VMEM_SHARED((NPAD,), jnp.float32),
        [pltpu.SemaphoreType.DMA] * 8,
        [pltpu.SemaphoreType.DMA] * 2,
        [pltpu.SemaphoreType.DMA] * 2,
        [pltpu.SemaphoreType.DMA] * 2,
        [pltpu.SemaphoreType.DMA] * 2,
    ],
    compiler_params=_sc_params,
)
def _agg_kernel(y0, y1, src3, dst3, dinv, zvec,
                agg0, agg1, t0, t1,
                sidx, didx, rows0, rows1, dvals0, dvals1, zvec_v,
                agg_sh, t_sh, isem, gsem, ssem, dgsem, tsem):
    c = lax.axis_index("c")
    s = lax.axis_index("s")
    off = s * ROWS_PER_TILE
    rows = [rows0, rows1]
    dvals = [dvals0, dvals1]

    for k in range(ROWS_PER_TILE // CHUNK):
        sl = pl.ds(off + k * CHUNK, CHUNK)

        @pl.when(c == 0)
        def _():
            pltpu.sync_copy(y0.at[sl], rows0)
            pltpu.sync_copy(rows0, agg_sh.at[sl])

        @pl.when(c == 1)
        def _():
            pltpu.sync_copy(y1.at[sl], rows0)
            pltpu.sync_copy(rows0, agg_sh.at[sl])

    pltpu.sync_copy(zvec, zvec_v)
    pltpu.sync_copy(zvec_v, t_sh.at[pl.ds(off, ROWS_PER_TILE)])

    plsc.subcore_barrier()

    def fire_idx(g, b8):
        pltpu.async_copy(src3.at[s, g], sidx.at[b8], isem[b8])
        pltpu.async_copy(dst3.at[s, g], didx.at[b8], isem[b8])

    def wait_idx(b8):
        pltpu.make_async_copy(src3.at[s, 0], sidx.at[b8], isem[b8]).wait()
        pltpu.make_async_copy(src3.at[s, 0], didx.at[b8], isem[b8]).wait()

    def fire_gather(b8, b2):
        @pl.when(c == 0)
        def _():
            pltpu.async_copy(y0.at[sidx.at[b8]], rows[b2], gsem[b2])

        @pl.when(c == 1)
        def _():
            pltpu.async_copy(y1.at[sidx.at[b8]], rows[b2], gsem[b2])

    def wait_rows(sem):
        pltpu.make_async_copy(y0.at[pl.ds(0, CHUNK)], rows0, sem).wait()

    def wait_dvals(sem, p):
        pltpu.make_async_copy(zvec.at[pl.ds(0, CHUNK)], dvals[p], sem).wait()

    def do_t(g):
        return jnp.where(c == 0, g < T_SPLIT, g >= T_SPLIT)

    fire_idx(0, 0)
    fire_idx(1, 1)
    wait_idx(0)
    fire_gather(0, 0)

    @pl.when(c == 0)
    def _():
        pltpu.async_copy(dinv.at[didx.at[0]], dvals[0], dgsem[0])

    @pl.loop(0, NCHUNK // 8)
    def _(G):
        for u in range(8):
            g = G * 8 + u
            b2 = u % 2
            p = u % 2
            pn = (u + 1) % 2
            nb8 = (u + 1) % 8

            wait_rows(gsem[b2])
            pltpu.async_copy(rows[b2], agg_sh.at[didx.at[u]], ssem[b2],
                             add=True)

            @pl.when(do_t(g))
            def _():
                wait_dvals(dgsem[p], p)
                pltpu.async_copy(dvals[p], t_sh.at[sidx.at[u]], tsem[p],
                                 add=True)

            @pl.when(g + 1 < NCHUNK)
            def _():
                wait_idx(nb8)

                @pl.when(do_t(g + 1))
                def _():
                    @pl.when(jnp.where(c == 0, g >= 1, g >= T_SPLIT + 1))
                    def _():
                        wait_dvals(tsem[pn], pn)

                    pltpu.async_copy(dinv.at[didx.at[nb8]], dvals[pn],
                                     dgsem[pn])

                @pl.when(g >= 1)
                def _():
                    wait_rows(ssem[pn])

                fire_gather(nb8, pn)

            @pl.when(g + 2 < NCHUNK)
            def _():
                fire_idx(g + 2, (u + 2) % 8)

    wait_rows(ssem[0])
    wait_rows(ssem[1])
    wait_dvals(tsem[0], 0)
    wait_dvals(tsem[1], 1)

    plsc.subcore_barrier()

    @pl.when(c == 0)
    def _():
        for k in range(ROWS_PER_TILE // CHUNK):
            o = off + k * CHUNK
            pltpu.sync_copy(agg_sh.at[pl.ds(o, CHUNK)], agg0.at[pl.ds(o, CHUNK)])
        pltpu.sync_copy(t_sh.at[pl.ds(off, ROWS_PER_TILE)],
                        t0.at[pl.ds(off, ROWS_PER_TILE)])

    @pl.when(c == 1)
    def _():
        for k in range(ROWS_PER_TILE // CHUNK):
            o = off + k * CHUNK
            pltpu.sync_copy(agg_sh.at[pl.ds(o, CHUNK)], agg1.at[pl.ds(o, CHUNK)])
        pltpu.sync_copy(t_sh.at[pl.ds(off, ROWS_PER_TILE)],
                        t1.at[pl.ds(off, ROWS_PER_TILE)])


def _layer1_body(x_ref, w_ref, deg_ref, y0_ref, y1_ref, dinv_ref):
    i = pl.program_id(0)
    rows = lax.broadcasted_iota(jnp.int32, (BR, 1), 0) + i * BR
    dinv = jnp.where(rows < N_NODES, lax.rsqrt(deg_ref[...] + 1.0), 0.0)
    h = jnp.dot(x_ref[...], w_ref[...], preferred_element_type=jnp.float32,
                precision=lax.Precision.DEFAULT)
    y = h * dinv
    y0_ref[...] = y[:, :128]
    y1_ref[...] = y[:, 128:]
    dinv_ref[...] = dinv


def _layer1(x, w_in, deg_col):
    return pl.pallas_call(
        _layer1_body,
        grid=(NPAD // BR,),
        in_specs=[
            pl.BlockSpec((BR, IN_DIM), lambda i: (i, 0)),
            pl.BlockSpec((IN_DIM, HID_DIM), lambda i: (0, 0)),
            pl.BlockSpec((BR, 1), lambda i: (i, 0)),
        ],
        out_specs=[
            pl.BlockSpec((BR, 128), lambda i: (i, 0)),
            pl.BlockSpec((BR, 128), lambda i: (i, 0)),
            pl.BlockSpec((BR, 1), lambda i: (i, 0)),
        ],
        out_shape=[
            jax.ShapeDtypeStruct((NPAD, 128), jnp.float32),
            jax.ShapeDtypeStruct((NPAD, 128), jnp.float32),
            jax.ShapeDtypeStruct((NPAD, 1), jnp.float32),
        ],
    )(x, w_in, deg_col)


def _final_body(agg0_ref, agg1_ref, dinv_ref, t0_ref, t1_ref,
                b_in_ref, w_out_ref, b_out_ref, out_ref, acc):
    i = pl.program_id(0)
    dinv = dinv_ref[...]
    cvec = dinv * (dinv + t0_ref[...] + t1_ref[...])
    pre = jnp.concatenate([agg0_ref[...], agg1_ref[...]], axis=1)
    x1 = jnp.maximum(pre * dinv + b_in_ref[...], 0.0)
    v = jnp.sum(x1 * cvec, axis=0, keepdims=True)

    @pl.when(i == 0)
    def _():
        acc[...] = v

    @pl.when(i > 0)
    def _():
        acc[...] += v

    @pl.when(i == NPAD // BR - 1)
    def _():
        out_ref[...] = (
            jnp.dot(acc[...], w_out_ref[...], preferred_element_type=jnp.float32,
                    precision=lax.Precision.HIGHEST) * (1.0 / N_NODES)
            + b_out_ref[...])


def _final(agg0, agg1, dinv_col, t0_col, t1_col, b_in, w_out, b_out):
    return pl.pallas_call(
        _final_body,
        grid=(NPAD // BR,),
        in_specs=[
            pl.BlockSpec((BR, 128), lambda i: (i, 0)),
            pl.BlockSpec((BR, 128), lambda i: (i, 0)),
            pl.BlockSpec((BR, 1), lambda i: (i, 0)),
            pl.BlockSpec((BR, 1), lambda i: (i, 0)),
            pl.BlockSpec((BR, 1), lambda i: (i, 0)),
            pl.BlockSpec((1, HID_DIM), lambda i: (0, 0)),
            pl.BlockSpec((HID_DIM, OUT_DIM), lambda i: (0, 0)),
            pl.BlockSpec((1, OUT_DIM), lambda i: (0, 0)),
        ],
        out_specs=pl.BlockSpec((1, OUT_DIM), lambda i: (0, 0)),
        out_shape=jax.ShapeDtypeStruct((1, OUT_DIM), jnp.float32),
        scratch_shapes=[pltpu.VMEM((1, HID_DIM), jnp.float32)],
    )(agg0, agg1, dinv_col, t0_col, t1_col, b_in, w_out, b_out)


def kernel(node_ids, edge_index, embed, W_in, b_in, W_out, b_out):
    src = edge_index[0].astype(jnp.int32)
    dst = edge_index[1].astype(jnp.int32)
    ids3 = jnp.concatenate(
        [node_ids.astype(jnp.int32), jnp.zeros((NPAD - N_NODES,), jnp.int32)]
    ).reshape(16, GCH, CHUNK)
    src3 = jnp.concatenate(
        [src, jnp.zeros((EPAD - N_EDGES,), jnp.int32)]).reshape(16, NCHUNK, CHUNK)
    dst3 = jnp.concatenate(
        [dst, jnp.full((EPAD - N_EDGES,), NPAD - 1, jnp.int32)]
    ).reshape(16, NCHUNK, CHUNK)
    ones = jnp.ones((CHUNK,), jnp.float32)
    zvec = jnp.zeros((ROWS_PER_TILE,), jnp.float32)

    x, deg = _gather_deg_kernel(embed, ids3, dst3, ones, zvec)
    y0, y1, dinv_col = _layer1(x, W_in, deg.reshape(NPAD, 1))
    agg0, agg1, t0, t1 = _agg_kernel(
        y0, y1, src3, dst3, dinv_col.reshape(NPAD), zvec)
    out = _final(agg0, agg1, dinv_col,
                 t0.reshape(NPAD, 1), t1.reshape(NPAD, 1),
                 b_in.reshape(1, HID_DIM), W_out, b_out.reshape(1, OUT_DIM))
    return out.reshape(OUT_DIM)
